# baseline (speedup 1.0000x reference)
import functools

import jax
import jax.numpy as jnp
from jax import lax
from jax.experimental import pallas as pl
from jax.experimental.pallas import tpu as pltpu
from jax.experimental.pallas import tpu_sc as plsc

_G = 16
_EPS = 1e-8
_SCALE = 0.5
_N = 100_000
_E = 3_200_000
_D = 128

_NC = 2
_NS = 16
_NT = _NC * _NS
_EPT = _E // _NT
_CH = 10_000
_NCHUNK = _EPT // _CH
_NPACK = _N // 8

_NODE_BLK = 2_000
_NODE_GRID = _N // _NODE_BLK


_NB = 5
_VPC = _CH // 16
_VSTEPS = _VPC // _NB


def _sc_edge_body(gi_hbm, src_hbm, d_hbm, out_hbm, table, srcb0, srcb1,
                  db0, db1, acc_c, acc_s, acc_q, obuf):
  wid = lax.axis_index("c") * _NS + lax.axis_index("s")
  base = wid * _EPT
  srcbs = (srcb0, srcb1)
  dbs = (db0, db1)

  pltpu.sync_copy(gi_hbm, table)
  for r in range(_NB * 16):
    acc_c[r] = jnp.zeros((16,), jnp.float32)
    acc_s[r] = jnp.zeros((16,), jnp.float32)
    acc_q[r] = jnp.zeros((16,), jnp.float32)
  lanes = lax.iota(jnp.int32, 16)
  ones16 = jnp.ones((16,), jnp.float32)

  for ci in range(_NCHUNK):
    off = base + ci * _CH
    buf = ci % 2
    sb = srcbs[buf]
    dbuf = dbs[buf]
    pltpu.sync_copy(src_hbm.at[pl.ds(off, _CH)], sb)
    pltpu.sync_copy(d_hbm.at[pl.ds(off, _CH)], dbuf)

    def vec_body(j, c2, sb=sb, dbuf=dbuf):
      o0 = j * (_NB * 16)
      for b in range(_NB):
        s = sb[pl.ds(o0 + b * 16, 16)]
        dv = dbuf[pl.ds(o0 + b * 16, 16)]
        w = plsc.load_gather(table, [lax.shift_right_logical(s, 3)])
        sh = lax.shift_left(lax.bitwise_and(s, 7), 2)
        g = lax.bitwise_and(lax.shift_right_logical(w, sh), 15) + (b * 16)
        plsc.addupdate_scatter(acc_c, [g, lanes], ones16)
        plsc.addupdate_scatter(acc_s, [g, lanes], dv)
        plsc.addupdate_scatter(acc_q, [g, lanes], dv * dv)
      return c2

    lax.fori_loop(0, _VSTEPS, vec_body, 0)

  for r in range(16):
    for j, a in enumerate((acc_c, acc_s, acc_q)):
      v = a[r]
      for b in range(1, _NB):
        v = v + a[b * 16 + r]
      obuf[j * 16 + r] = v
  pltpu.sync_copy(obuf, out_hbm.at[wid])


@functools.cache
def _sc_edge():
  return functools.partial(
      pl.kernel,
      mesh=plsc.VectorSubcoreMesh(core_axis_name="c", subcore_axis_name="s"),
      compiler_params=pltpu.CompilerParams(
          needs_layout_passes=False, disable_bounds_checks=True),
      out_type=jax.ShapeDtypeStruct((_NT, 48, 16), jnp.float32),
      scratch_types=[
          pltpu.VMEM((_NPACK,), jnp.int32),
          pltpu.VMEM((_CH,), jnp.int32),
          pltpu.VMEM((_CH,), jnp.int32),
          pltpu.VMEM((_CH,), jnp.float32),
          pltpu.VMEM((_CH,), jnp.float32),
          pltpu.VMEM((_NB * 16, 16), jnp.float32),
          pltpu.VMEM((_NB * 16, 16), jnp.float32),
          pltpu.VMEM((_NB * 16, 16), jnp.float32),
          pltpu.VMEM((48, 16), jnp.float32),
      ],
  )(_sc_edge_body)


def _tc_node_body(x_ref, gi_ref, out_ref, acc):
  i = pl.program_id(0)

  @pl.when(i == 0)
  def _init():
    acc[...] = jnp.zeros_like(acc)

  gi = gi_ref[...].reshape(1, _NODE_BLK)
  g_iota = lax.broadcasted_iota(jnp.int32, (_G, _NODE_BLK), 0)
  oh = (g_iota == gi).astype(jnp.float32)
  part = jnp.dot(oh, x_ref[...], preferred_element_type=jnp.float32)
  cntp = jnp.sum(oh, axis=1, keepdims=True)
  acc[:, :_D] += part
  acc[:, _D:_D + 1] += cntp

  @pl.when(i == _NODE_GRID - 1)
  def _done():
    out_ref[...] = acc[...]


def _tc_node(x20, gi3):
  return pl.pallas_call(
      _tc_node_body,
      grid=(_NODE_GRID,),
      in_specs=[
          pl.BlockSpec((_NODE_BLK, _D), lambda i: (i, 0)),
          pl.BlockSpec((1, 1, _NODE_BLK), lambda i: (i, 0, 0)),
      ],
      out_specs=pl.BlockSpec((_G, 256), lambda i: (0, 0)),
      out_shape=jax.ShapeDtypeStruct((_G, 256), jnp.float32),
      scratch_shapes=[pltpu.VMEM((_G, 256), jnp.float32)],
  )(x20, gi3)


def _gelu(x):
  return x * 0.5 * (1.0 + lax.erf(x / jnp.sqrt(2.0).astype(jnp.float32)))


def _film(o, dsz):
  gamma = 1.0 + _SCALE * jnp.tanh(o[:, :dsz])
  beta = _SCALE * jnp.tanh(o[:, dsz:])
  return gamma, beta


def _tc_mlp_body(node_ref, edge_ref, w1a_ref, w1b_ref, b1_ref, w2_ref, b2_ref,
                 wg_ref, bgw_ref, wt_ref, btw_ref, wf_ref, bfw_ref,
                 gg_ref, bg_ref, gt_ref, bt_ref, gf_ref, bf_ref):
  node = node_ref[...]
  cnt = node[:, _D:_D + 1]
  mean = node[:, :_D] / (cnt + _EPS)
  logn = jnp.log1p(cnt)

  er = edge_ref[...]
  ls = jnp.dot(er, jnp.ones((16, 1), jnp.float32),
               preferred_element_type=jnp.float32)
  j_iota = lax.broadcasted_iota(jnp.int32, (48, _NT * 48), 1)
  r_iota = lax.broadcasted_iota(jnp.int32, (48, _NT * 48), 0)
  sel = (j_iota % 48 == r_iota).astype(jnp.float32)
  stats = jnp.dot(sel, ls, preferred_element_type=jnp.float32)
  ec = stats[0:16, :]
  sd = stats[16:32, :]
  sq = stats[32:48, :]
  em = sd / (ec + _EPS)
  em2 = sq / (ec + _EPS)
  var = jnp.clip(em2 - em * em, 0.0, None)
  es = jnp.sqrt(var + _EPS)

  feats = jnp.concatenate(
      [logn, em, es, jnp.zeros((_G, 5), jnp.float32)], axis=1)
  h = jnp.dot(mean, w1a_ref[...], preferred_element_type=jnp.float32)
  h += jnp.dot(feats, w1b_ref[...], preferred_element_type=jnp.float32)
  h = _gelu(h + b1_ref[...])
  h = _gelu(jnp.dot(h, w2_ref[...], preferred_element_type=jnp.float32)
            + b2_ref[...])

  og = jnp.dot(h, wg_ref[...], preferred_element_type=jnp.float32) + bgw_ref[...]
  ot = jnp.dot(h, wt_ref[...], preferred_element_type=jnp.float32) + btw_ref[...]
  of = jnp.dot(h, wf_ref[...], preferred_element_type=jnp.float32) + bfw_ref[...]
  gg_ref[...], bg_ref[...] = _film(og, 128)
  gt_ref[...], bt_ref[...] = _film(ot, 256)
  gf_ref[...], bf_ref[...] = _film(of, 512)


def _tc_mlp(node, edge2d, w1a, w1b, b1, w2, b2, wg, bgw, wt, btw, wf, bfw):
  out_shapes = (
      jax.ShapeDtypeStruct((_G, 128), jnp.float32),
      jax.ShapeDtypeStruct((_G, 128), jnp.float32),
      jax.ShapeDtypeStruct((_G, 256), jnp.float32),
      jax.ShapeDtypeStruct((_G, 256), jnp.float32),
      jax.ShapeDtypeStruct((_G, 512), jnp.float32),
      jax.ShapeDtypeStruct((_G, 512), jnp.float32),
  )
  return pl.pallas_call(_tc_mlp_body, out_shape=out_shapes)(
      node, edge2d, w1a, w1b, b1, w2, b2, wg, bgw, wt, btw, wf, bfw)


def kernel(x20, edge_attr, W1, b1, W2, b2, Wg, bgw, Wt, btw, Wf, bfw,
           graph_idx, edge_index):
  src = edge_index[0]
  d = edge_attr.reshape(-1).astype(jnp.float32)
  gi3 = graph_idx.reshape(_NODE_GRID, 1, _NODE_BLK)

  gi8 = graph_idx.astype(jnp.int32).reshape(_NPACK, 8)
  packed = gi8[:, 0]
  for k in range(1, 8):
    packed = jnp.bitwise_or(packed, jnp.left_shift(gi8[:, k], 4 * k))

  edge_part = _sc_edge()(packed, src, d)
  node_agg = _tc_node(x20, gi3)

  w1a = W1[:_D, :]
  w1b = jnp.zeros((8, _D), jnp.float32).at[:3, :].set(W1[_D:, :])
  outs = _tc_mlp(node_agg, edge_part.reshape(_NT * 48, 16),
                 w1a, w1b, b1.reshape(1, -1), W2, b2.reshape(1, -1),
                 Wg, bgw.reshape(1, -1), Wt, btw.reshape(1, -1),
                 Wf, bfw.reshape(1, -1))
  return outs


# speedup vs baseline: 1.4962x; 1.4962x over previous
import functools

import jax
import jax.numpy as jnp
from jax import lax
from jax.experimental import pallas as pl
from jax.experimental.pallas import tpu as pltpu
from jax.experimental.pallas import tpu_sc as plsc

_G = 16
_EPS = 1e-8
_SCALE = 0.5
_N = 100_000
_E = 3_200_000
_D = 128

_NC = 2
_NS = 16
_NT = _NC * _NS
_EPT = _E // _NT
_CH = 10_000
_NCHUNK = _EPT // _CH
_NPACK = _N // 8

_NODE_BLK = 2_000
_NODE_GRID = _N // _NODE_BLK


_NB = 5
_VPC = _CH // 16
_VSTEPS = _VPC // _NB


def _sc_edge_body(gi_hbm, src_hbm, d_hbm, out_hbm, table, srcb0, srcb1,
                  db0, db1, acc_c, acc_s, acc_q, obuf, sem0, sem1):
  wid = lax.axis_index("c") * _NS + lax.axis_index("s")
  base = wid * _EPT
  srcbs = (srcb0, srcb1)
  dbs = (db0, db1)
  sems = (sem0, sem1)

  def start_chunk(ci):
    off = base + ci * _CH
    buf = ci % 2
    return (
        pltpu.async_copy(src_hbm.at[pl.ds(off, _CH)], srcbs[buf], sems[buf]),
        pltpu.async_copy(d_hbm.at[pl.ds(off, _CH)], dbs[buf], sems[buf]),
    )

  pending = start_chunk(0)
  pltpu.sync_copy(gi_hbm, table)
  for r in range(_NB * 16):
    acc_c[r] = jnp.zeros((16,), jnp.float32)
    acc_s[r] = jnp.zeros((16,), jnp.float32)
    acc_q[r] = jnp.zeros((16,), jnp.float32)
  lanes = lax.iota(jnp.int32, 16)
  ones16 = jnp.ones((16,), jnp.float32)

  for ci in range(_NCHUNK):
    nxt = start_chunk(ci + 1) if ci + 1 < _NCHUNK else None
    for cp in pending:
      cp.wait()
    pending = nxt
    buf = ci % 2
    sb = srcbs[buf]
    dbuf = dbs[buf]

    def vec_body(j, c2, sb=sb, dbuf=dbuf):
      o0 = j * (_NB * 16)
      ss = [sb[pl.ds(o0 + b * 16, 16)] for b in range(_NB)]
      dvs = [dbuf[pl.ds(o0 + b * 16, 16)] for b in range(_NB)]
      ws = [plsc.load_gather(table, [lax.shift_right_logical(s, 3)])
            for s in ss]
      shs = [lax.shift_left(lax.bitwise_and(s, 7), 2) for s in ss]
      gs = [lax.bitwise_and(lax.shift_right_logical(w, sh), 15) + (b * 16)
            for b, (w, sh) in enumerate(zip(ws, shs))]
      sqs = [dv * dv for dv in dvs]
      for b in range(_NB):
        plsc.addupdate_scatter(acc_c, [gs[b], lanes], ones16)
        plsc.addupdate_scatter(acc_s, [gs[b], lanes], dvs[b])
        plsc.addupdate_scatter(acc_q, [gs[b], lanes], sqs[b])
      return c2

    lax.fori_loop(0, _VSTEPS, vec_body, 0)

  for r in range(16):
    for j, a in enumerate((acc_c, acc_s, acc_q)):
      v = a[r]
      for b in range(1, _NB):
        v = v + a[b * 16 + r]
      obuf[j * 16 + r] = v
  pltpu.sync_copy(obuf, out_hbm.at[wid])


@functools.cache
def _sc_edge():
  return functools.partial(
      pl.kernel,
      mesh=plsc.VectorSubcoreMesh(core_axis_name="c", subcore_axis_name="s"),
      compiler_params=pltpu.CompilerParams(
          needs_layout_passes=False, disable_bounds_checks=True),
      out_type=jax.ShapeDtypeStruct((_NT, 48, 16), jnp.float32),
      scratch_types=[
          pltpu.VMEM((_NPACK,), jnp.int32),
          pltpu.VMEM((_CH,), jnp.int32),
          pltpu.VMEM((_CH,), jnp.int32),
          pltpu.VMEM((_CH,), jnp.float32),
          pltpu.VMEM((_CH,), jnp.float32),
          pltpu.VMEM((_NB * 16, 16), jnp.float32),
          pltpu.VMEM((_NB * 16, 16), jnp.float32),
          pltpu.VMEM((_NB * 16, 16), jnp.float32),
          pltpu.VMEM((48, 16), jnp.float32),
          pltpu.SemaphoreType.DMA,
          pltpu.SemaphoreType.DMA,
      ],
  )(_sc_edge_body)


def _tc_node_body(x_ref, gi_ref, out_ref, acc):
  i = pl.program_id(0)

  @pl.when(i == 0)
  def _init():
    acc[...] = jnp.zeros_like(acc)

  gi = gi_ref[...].reshape(1, _NODE_BLK)
  g_iota = lax.broadcasted_iota(jnp.int32, (_G, _NODE_BLK), 0)
  oh = (g_iota == gi).astype(jnp.float32)
  part = jnp.dot(oh, x_ref[...], preferred_element_type=jnp.float32)
  cntp = jnp.sum(oh, axis=1, keepdims=True)
  acc[:, :_D] += part
  acc[:, _D:_D + 1] += cntp

  @pl.when(i == _NODE_GRID - 1)
  def _done():
    out_ref[...] = acc[...]


def _tc_node(x20, gi3):
  return pl.pallas_call(
      _tc_node_body,
      grid=(_NODE_GRID,),
      in_specs=[
          pl.BlockSpec((_NODE_BLK, _D), lambda i: (i, 0)),
          pl.BlockSpec((1, 1, _NODE_BLK), lambda i: (i, 0, 0)),
      ],
      out_specs=pl.BlockSpec((_G, 256), lambda i: (0, 0)),
      out_shape=jax.ShapeDtypeStruct((_G, 256), jnp.float32),
      scratch_shapes=[pltpu.VMEM((_G, 256), jnp.float32)],
  )(x20, gi3)


def _gelu(x):
  return x * 0.5 * (1.0 + lax.erf(x / jnp.sqrt(2.0).astype(jnp.float32)))


def _film(o, dsz):
  gamma = 1.0 + _SCALE * jnp.tanh(o[:, :dsz])
  beta = _SCALE * jnp.tanh(o[:, dsz:])
  return gamma, beta


def _tc_mlp_body(node_ref, edge_ref, w1a_ref, w1b_ref, b1_ref, w2_ref, b2_ref,
                 wg_ref, bgw_ref, wt_ref, btw_ref, wf_ref, bfw_ref,
                 gg_ref, bg_ref, gt_ref, bt_ref, gf_ref, bf_ref):
  node = node_ref[...]
  cnt = node[:, _D:_D + 1]
  mean = node[:, :_D] / (cnt + _EPS)
  logn = jnp.log1p(cnt)

  er = edge_ref[...]
  ls = jnp.dot(er, jnp.ones((16, 1), jnp.float32),
               preferred_element_type=jnp.float32)
  j_iota = lax.broadcasted_iota(jnp.int32, (48, _NT * 48), 1)
  r_iota = lax.broadcasted_iota(jnp.int32, (48, _NT * 48), 0)
  sel = (j_iota % 48 == r_iota).astype(jnp.float32)
  stats = jnp.dot(sel, ls, preferred_element_type=jnp.float32)
  ec = stats[0:16, :]
  sd = stats[16:32, :]
  sq = stats[32:48, :]
  em = sd / (ec + _EPS)
  em2 = sq / (ec + _EPS)
  var = jnp.clip(em2 - em * em, 0.0, None)
  es = jnp.sqrt(var + _EPS)

  feats = jnp.concatenate(
      [logn, em, es, jnp.zeros((_G, 5), jnp.float32)], axis=1)
  h = jnp.dot(mean, w1a_ref[...], preferred_element_type=jnp.float32)
  h += jnp.dot(feats, w1b_ref[...], preferred_element_type=jnp.float32)
  h = _gelu(h + b1_ref[...])
  h = _gelu(jnp.dot(h, w2_ref[...], preferred_element_type=jnp.float32)
            + b2_ref[...])

  og = jnp.dot(h, wg_ref[...], preferred_element_type=jnp.float32) + bgw_ref[...]
  ot = jnp.dot(h, wt_ref[...], preferred_element_type=jnp.float32) + btw_ref[...]
  of = jnp.dot(h, wf_ref[...], preferred_element_type=jnp.float32) + bfw_ref[...]
  gg_ref[...], bg_ref[...] = _film(og, 128)
  gt_ref[...], bt_ref[...] = _film(ot, 256)
  gf_ref[...], bf_ref[...] = _film(of, 512)


def _tc_mlp(node, edge2d, w1a, w1b, b1, w2, b2, wg, bgw, wt, btw, wf, bfw):
  out_shapes = (
      jax.ShapeDtypeStruct((_G, 128), jnp.float32),
      jax.ShapeDtypeStruct((_G, 128), jnp.float32),
      jax.ShapeDtypeStruct((_G, 256), jnp.float32),
      jax.ShapeDtypeStruct((_G, 256), jnp.float32),
      jax.ShapeDtypeStruct((_G, 512), jnp.float32),
      jax.ShapeDtypeStruct((_G, 512), jnp.float32),
  )
  return pl.pallas_call(_tc_mlp_body, out_shape=out_shapes)(
      node, edge2d, w1a, w1b, b1, w2, b2, wg, bgw, wt, btw, wf, bfw)


def kernel(x20, edge_attr, W1, b1, W2, b2, Wg, bgw, Wt, btw, Wf, bfw,
           graph_idx, edge_index):
  src = edge_index[0]
  d = edge_attr.reshape(-1).astype(jnp.float32)
  gi3 = graph_idx.reshape(_NODE_GRID, 1, _NODE_BLK)

  gi8 = graph_idx.astype(jnp.int32).reshape(_NPACK, 8)
  packed = gi8[:, 0]
  for k in range(1, 8):
    packed = jnp.bitwise_or(packed, jnp.left_shift(gi8[:, k], 4 * k))

  edge_part = _sc_edge()(packed, src, d)
  node_agg = _tc_node(x20, gi3)

  w1a = W1[:_D, :]
  w1b = jnp.zeros((8, _D), jnp.float32).at[:3, :].set(W1[_D:, :])
  outs = _tc_mlp(node_agg, edge_part.reshape(_NT * 48, 16),
                 w1a, w1b, b1.reshape(1, -1), W2, b2.reshape(1, -1),
                 Wg, bgw.reshape(1, -1), Wt, btw.reshape(1, -1),
                 Wf, bfw.reshape(1, -1))
  return outs
